# baseline (speedup 1.0000x reference)
import functools

import jax
import jax.numpy as jnp
from jax import lax
from jax.experimental import pallas as pl
from jax.experimental.pallas import tpu as pltpu

INPUT_DIM = 1024
HIDDEN_DIM = 8192
OUTPUT_DIM = 1024
SPARSITY = 2
ALPHA = 5.0
BETA = 0.5

_HT = 512
_NSTEP = HIDDEN_DIM // _HT
_TINY = 1.1754943508222875e-38
_KS1 = 42
_KS2 = 0x1BD11BDA ^ 42

_ROT = ((13, 15, 26, 6), (17, 29, 16, 24))


def _rotl(v, r):
    return lax.shift_left(v, r) | lax.shift_right_logical(v, 32 - r)


def _threefry_bits(j):
    ks = (0, _KS1, _KS2)
    x0 = jnp.zeros_like(j)
    x1 = j + jnp.int32(_KS1)
    for i in range(5):
        for r in _ROT[i % 2]:
            x0 = x0 + x1
            x1 = _rotl(x1, r) ^ x0
        x0 = x0 + jnp.int32(ks[(i + 1) % 3])
        x1 = x1 + jnp.int32(ks[(i + 2) % 3] + (i + 1))
    return x0 ^ x1


def _gumbel_from_bits(bits):
    fb = lax.shift_right_logical(bits, 9) | jnp.int32(0x3F800000)
    f = lax.bitcast_convert_type(fb, jnp.float32) - jnp.float32(1.0)
    u = jnp.maximum(_TINY, f + _TINY)
    return -jnp.log(-jnp.log(u))


def _sample_body(x_ref, w_ref, b_ref, idx0_ref, idx1_ref, corr_ref,
                 m_ref, a_ref, ps_ref):
    k = pl.program_id(0)
    h0 = k * _HT
    nb = x_ref.shape[0]

    @pl.when(k == 0)
    def _init():
        m_ref[...] = jnp.full_like(m_ref, -jnp.inf)
        a_ref[...] = jnp.zeros_like(a_ref)
        ps_ref[...] = jnp.zeros_like(ps_ref)

    z = lax.dot_general(x_ref[...], w_ref[...],
                        (((1,), (1,)), ((), ())),
                        preferred_element_type=jnp.float32)
    z = z + b_ref[...][None, :]
    p = 1.0 / (1.0 + jnp.exp(-ALPHA * (z - BETA)))
    ps_ref[...] += jnp.sum(p, axis=1, keepdims=True)
    logit = jnp.log(p + 1e-30)

    rowbase = lax.broadcasted_iota(jnp.int32, (nb, _HT), 0) * jnp.int32(
        SPARSITY * HIDDEN_DIM)
    col = lax.broadcasted_iota(jnp.int32, (nb, _HT), 1)
    cols_f = col.astype(jnp.float32)
    for s in range(SPARSITY):
        j = rowbase + (col + jnp.int32(s * HIDDEN_DIM + h0))
        v = logit + _gumbel_from_bits(_threefry_bits(j))
        tmax = jnp.max(v, axis=1, keepdims=True)
        targ = jnp.min(jnp.where(v == tmax, cols_f, jnp.float32(HIDDEN_DIM)),
                       axis=1, keepdims=True)
        upd = tmax > m_ref[:, s:s + 1]
        m_ref[:, s:s + 1] = jnp.where(upd, tmax, m_ref[:, s:s + 1])
        a_ref[:, s:s + 1] = jnp.where(
            upd, targ.astype(jnp.int32) + jnp.int32(h0), a_ref[:, s:s + 1])

    @pl.when(k == _NSTEP - 1)
    def _fin():
        idx0_ref[...] = a_ref[:, 0]
        idx1_ref[...] = a_ref[:, 1]
        corr_ref[...] = jnp.broadcast_to(
            ps_ref[...] / SPARSITY, corr_ref.shape)


def _tc_sample(xf, W_in, b_in):
    nb = xf.shape[0]
    return pl.pallas_call(
        _sample_body,
        grid=(_NSTEP,),
        in_specs=[
            pl.BlockSpec((nb, INPUT_DIM), lambda k: (0, 0)),
            pl.BlockSpec((_HT, INPUT_DIM), lambda k: (k, 0)),
            pl.BlockSpec((_HT,), lambda k: (k,)),
        ],
        out_specs=[
            pl.BlockSpec((nb,), lambda k: (0,)),
            pl.BlockSpec((nb,), lambda k: (0,)),
            pl.BlockSpec((nb, 16), lambda k: (0, 0)),
        ],
        out_shape=[
            jax.ShapeDtypeStruct((nb,), jnp.int32),
            jax.ShapeDtypeStruct((nb,), jnp.int32),
            jax.ShapeDtypeStruct((nb, 16), jnp.float32),
        ],
        scratch_shapes=[
            pltpu.VMEM((nb, SPARSITY), jnp.float32),
            pltpu.VMEM((nb, SPARSITY), jnp.int32),
            pltpu.VMEM((nb, 1), jnp.float32),
        ],
    )(xf, W_in, b_in)


def _make_sc_gather(nb):
    from jax.experimental.pallas import tpu_sc as plsc

    info = plsc.get_sparse_core_info()
    nw = info.num_cores * info.num_subcores
    rows_per_w = nb // nw
    cb = 32
    nchunk = rows_per_w // cb
    mesh = plsc.VectorSubcoreMesh(core_axis_name="c", subcore_axis_name="s")

    @functools.partial(
        pl.kernel, mesh=mesh,
        out_type=jax.ShapeDtypeStruct((nb, OUTPUT_DIM), jnp.float32),
        scratch_types=[
            pltpu.VMEM((cb,), jnp.int32),
            pltpu.VMEM((cb,), jnp.int32),
            pltpu.VMEM((cb, OUTPUT_DIM), jnp.float32),
            pltpu.VMEM((cb, OUTPUT_DIM), jnp.float32),
            pltpu.VMEM((cb, 16), jnp.float32),
            pltpu.SemaphoreType.DMA,
            pltpu.SemaphoreType.DMA,
        ],
    )
    def sc_gather(idx0_hbm, idx1_hbm, corr_hbm, table_hbm, out_hbm,
                  idx0_v, idx1_v, rows0_v, rows1_v, corr_v, sem0, sem1):
        wid = lax.axis_index("s") * info.num_cores + lax.axis_index("c")
        base = wid * rows_per_w
        for c in range(nchunk):
            off = base + c * cb
            pltpu.sync_copy(idx0_hbm.at[pl.ds(off, cb)], idx0_v)
            pltpu.sync_copy(idx1_hbm.at[pl.ds(off, cb)], idx1_v)
            pltpu.sync_copy(corr_hbm.at[pl.ds(off, cb)], corr_v)
            cp0 = pltpu.async_copy(table_hbm.at[idx0_v], rows0_v, sem0)
            cp1 = pltpu.async_copy(table_hbm.at[idx1_v], rows1_v, sem1)
            cp0.wait()
            cp1.wait()

            def row_body(r, carry):
                cv = corr_v[r]

                def col_body(cc, carry2):
                    sl = pl.ds(cc * 16, 16)
                    a = rows0_v[r, sl]
                    b = rows1_v[r, sl]
                    rows0_v[r, sl] = (a + b) * cv
                    return carry2

                return lax.fori_loop(0, OUTPUT_DIM // 16, col_body, carry)

            lax.fori_loop(0, cb, row_body, 0)
            pltpu.sync_copy(rows0_v, out_hbm.at[pl.ds(off, cb)])

    return sc_gather


def kernel(x, W_in, b_in, out_weight):
    shape0 = x.shape[:-1]
    xf = x.reshape(-1, x.shape[-1])
    nb = xf.shape[0]
    idx0, idx1, corr_rep = _tc_sample(xf, W_in, b_in)
    out = _make_sc_gather(nb)(idx0, idx1, corr_rep, out_weight)
    return out.reshape(*shape0, OUTPUT_DIM)
